# Initial kernel scaffold; baseline (speedup 1.0000x reference)
import functools

import jax
import jax.numpy as jnp
from jax import lax
from jax.experimental import pallas as pl
from jax.experimental.pallas import tpu as pltpu
from jax.experimental.pallas import tpu_sc as plsc

N = 10000
E = 320000
D = 128
H = 64
O = 32
BN_EPS = 1e-5

NP = 10240
NC, NS = 2, 16
NW = NC * NS
CH = 128
EW = -(-E // NW)
K = -(-EW // CH)
EPAD = NW * K * CH
RPW = NP // NS
RB = 1024


def _make_seg_sum(width: int, with_cnt: bool):
  mesh = plsc.VectorSubcoreMesh(core_axis_name="c", subcore_axis_name="s")
  out_type = [jax.ShapeDtypeStruct((NC, NP, width), jnp.float32)]
  if with_cnt:
    out_type.append(jax.ShapeDtypeStruct((NC, NP), jnp.float32))
  scratch = [
      pltpu.VMEM((K, CH), jnp.int32),
      pltpu.VMEM((K, CH), jnp.int32),
      pltpu.VMEM((CH, width), jnp.float32),
      pltpu.VMEM((CH,), jnp.float32),
      pltpu.VMEM((CH,), jnp.float32),
      pltpu.VMEM_SHARED((NP, width), jnp.float32),
      pltpu.VMEM_SHARED((NP,), jnp.float32),
      pltpu.SemaphoreType.DMA,
  ]

  def body(table, srcw, dstw, ones, zrows, zvec, *rest):
    if with_cnt:
      parts, cnts, src_v, dst_v, rows_v, ones_v, zv_v, acc, cacc, sem = rest
    else:
      parts, src_v, dst_v, rows_v, ones_v, zv_v, acc, cacc, sem = rest
    sid = lax.axis_index("s")
    cid = lax.axis_index("c")
    wid = sid * NC + cid
    r0 = sid * RPW

    pltpu.sync_copy(zrows, rows_v)
    if with_cnt:
      pltpu.sync_copy(zvec, zv_v)
    for t in range(RPW // CH):
      base = r0 + t * CH
      pltpu.sync_copy(rows_v, acc.at[pl.ds(base, CH)])
      if with_cnt:
        pltpu.sync_copy(zv_v, cacc.at[pl.ds(base, CH)])
    if with_cnt:
      pltpu.sync_copy(ones, ones_v)
    pltpu.sync_copy(srcw.at[wid], src_v)
    pltpu.sync_copy(dstw.at[wid], dst_v)
    plsc.subcore_barrier()

    @pl.loop(0, K)
    def _edge_chunk(j):
      pltpu.async_copy(table.at[src_v.at[j]], rows_v, sem).wait()
      pltpu.sync_copy(rows_v, acc.at[dst_v.at[j]], add=True)
      if with_cnt:
        pltpu.sync_copy(ones_v, cacc.at[dst_v.at[j]], add=True)

    plsc.subcore_barrier()

    for t in range(RPW // CH):
      base = r0 + t * CH
      pltpu.sync_copy(acc.at[pl.ds(base, CH)], rows_v)
      pltpu.sync_copy(rows_v, parts.at[cid, pl.ds(base, CH)])
      if with_cnt:
        pltpu.sync_copy(cacc.at[pl.ds(base, CH)], zv_v)
        pltpu.sync_copy(zv_v, cnts.at[cid, pl.ds(base, CH)])

  return pl.kernel(body, out_type=tuple(out_type), mesh=mesh,
                   scratch_types=scratch)


_seg_sum_cnt = _make_seg_sum(H, with_cnt=True)
_seg_sum_o = _make_seg_sum(O, with_cnt=False)


def _tc_project(xp, wl, wr):
  def body(x_ref, wl_ref, wr_ref, p_ref, r_ref):
    xb = x_ref[...]
    p_ref[...] = jnp.dot(xb, wl_ref[...], preferred_element_type=jnp.float32)
    r_ref[...] = jnp.dot(xb, wr_ref[...], preferred_element_type=jnp.float32)

  return pl.pallas_call(
      body,
      grid=(NP // RB,),
      in_specs=[
          pl.BlockSpec((RB, D), lambda i: (i, 0)),
          pl.BlockSpec((D, H), lambda i: (0, 0)),
          pl.BlockSpec((D, H), lambda i: (0, 0)),
      ],
      out_specs=[
          pl.BlockSpec((RB, H), lambda i: (i, 0)),
          pl.BlockSpec((RB, H), lambda i: (i, 0)),
      ],
      out_shape=[
          jax.ShapeDtypeStruct((NP, H), jnp.float32),
          jax.ShapeDtypeStruct((NP, H), jnp.float32),
      ],
  )(xp, wl, wr)


def _tc_mid(parts0, cntt, r0, alpha, bb, wcat):
  def body(pp_ref, cn_ref, r0_ref, al_ref, bb_ref, w_ref, p1_ref, r1_ref):
    agg = pp_ref[0] + pp_ref[1]
    cnt = jnp.maximum(cn_ref[:, 0:1] + cn_ref[:, 1:2], 1.0)
    mean = agg / cnt
    h = jnp.maximum((mean + r0_ref[...]) * al_ref[...] + bb_ref[...], 0.0)
    pr = jnp.dot(h, w_ref[...], preferred_element_type=jnp.float32)
    p1_ref[...] = pr[:, :O]
    r1_ref[...] = pr[:, O:]

  return pl.pallas_call(
      body,
      grid=(NP // RB,),
      in_specs=[
          pl.BlockSpec((NC, RB, H), lambda i: (0, i, 0)),
          pl.BlockSpec((RB, NC), lambda i: (i, 0)),
          pl.BlockSpec((RB, H), lambda i: (i, 0)),
          pl.BlockSpec((1, H), lambda i: (0, 0)),
          pl.BlockSpec((1, H), lambda i: (0, 0)),
          pl.BlockSpec((H, 2 * O), lambda i: (0, 0)),
      ],
      out_specs=[
          pl.BlockSpec((RB, O), lambda i: (i, 0)),
          pl.BlockSpec((RB, O), lambda i: (i, 0)),
      ],
      out_shape=[
          jax.ShapeDtypeStruct((NP, O), jnp.float32),
          jax.ShapeDtypeStruct((NP, O), jnp.float32),
      ],
  )(parts0, cntt, r0, alpha, bb, wcat)


def _tc_final(parts1, cntt, r1, b1):
  def body(pp_ref, cn_ref, r1_ref, b1_ref, out_ref):
    agg = pp_ref[0] + pp_ref[1]
    cnt = jnp.maximum(cn_ref[:, 0:1] + cn_ref[:, 1:2], 1.0)
    out_ref[...] = agg / cnt + r1_ref[...] + b1_ref[...]

  return pl.pallas_call(
      body,
      grid=(NP // RB,),
      in_specs=[
          pl.BlockSpec((NC, RB, O), lambda i: (0, i, 0)),
          pl.BlockSpec((RB, NC), lambda i: (i, 0)),
          pl.BlockSpec((RB, O), lambda i: (i, 0)),
          pl.BlockSpec((1, O), lambda i: (0, 0)),
      ],
      out_specs=pl.BlockSpec((RB, O), lambda i: (i, 0)),
      out_shape=jax.ShapeDtypeStruct((NP, O), jnp.float32),
  )(parts1, cntt, r1, b1)


def kernel(x, edge_index, Wl0, Wr0, b0, gamma0, beta0, Wl1, Wr1, b1):
  f32 = jnp.float32
  xp = jnp.pad(x, ((0, NP - N), (0, 0)))
  src = jnp.concatenate(
      [edge_index[0], jnp.zeros((EPAD - E,), jnp.int32)]).reshape(NW, K, CH)
  dst = jnp.concatenate(
      [edge_index[1], jnp.full((EPAD - E,), NP - 1, jnp.int32)]).reshape(NW, K, CH)
  ones = jnp.ones((CH,), f32)
  zvec = jnp.zeros((CH,), f32)
  zrows_h = jnp.zeros((CH, H), f32)
  zrows_o = jnp.zeros((CH, O), f32)

  p0, r0 = _tc_project(xp, Wl0, Wr0)
  parts0, cntp = _seg_sum_cnt(p0, src, dst, ones, zrows_h, zvec)
  cntt = cntp.T

  scale = 1.0 / jnp.sqrt(jnp.float32(1.0) + BN_EPS)
  alpha = (gamma0 * scale).reshape(1, H)
  bb = (b0 * gamma0 * scale + beta0).reshape(1, H)
  wcat = jnp.concatenate([Wl1, Wr1], axis=1)

  p1, r1 = _tc_mid(parts0, cntt, r0, alpha, bb, wcat)
  parts1 = _seg_sum_o(p1, src, dst, ones, zrows_o, zvec)
  out = _tc_final(parts1, cntt, r1, b1.reshape(1, O))
  return out[:N]


# speedup vs baseline: 8.6199x; 8.6199x over previous
import functools

import jax
import jax.numpy as jnp
from jax import lax
from jax.experimental import pallas as pl
from jax.experimental.pallas import tpu as pltpu
from jax.experimental.pallas import tpu_sc as plsc

N = 10000
E = 320000
D = 128
H = 64
O = 32
BN_EPS = 1e-5

NP = 10240
NC, NS = 2, 16
NW = NC * NS
CH = 128
EW = -(-E // NW)
K = -(-EW // CH)
EPAD = NW * K * CH
RPW = NP // NS
RB = 1024


def _make_seg_sum(width: int, with_cnt: bool):
  mesh = plsc.VectorSubcoreMesh(core_axis_name="c", subcore_axis_name="s")
  out_type = [jax.ShapeDtypeStruct((NC, NP, width), jnp.float32)]
  if with_cnt:
    out_type.append(jax.ShapeDtypeStruct((NC, NP), jnp.float32))
  scratch = [
      pltpu.VMEM((K, CH), jnp.int32),
      pltpu.VMEM((K, CH), jnp.int32),
      pltpu.VMEM((CH, width), jnp.float32),
      pltpu.VMEM((CH,), jnp.float32),
      pltpu.VMEM((CH,), jnp.float32),
      pltpu.VMEM_SHARED((NP, width), jnp.float32),
      pltpu.VMEM_SHARED((NP,), jnp.float32),
      pltpu.SemaphoreType.DMA,
  ]

  def body(table, srcw, dstw, ones, zrows, zvec, *rest):
    if with_cnt:
      parts, cnts, src_v, dst_v, rows_v, ones_v, zv_v, acc, cacc, sem = rest
    else:
      parts, src_v, dst_v, rows_v, ones_v, zv_v, acc, cacc, sem = rest
    sid = lax.axis_index("s")
    cid = lax.axis_index("c")
    wid = sid * NC + cid
    r0 = sid * RPW

    pltpu.sync_copy(zrows, rows_v)
    if with_cnt:
      pltpu.sync_copy(zvec, zv_v)
    for t in range(RPW // CH):
      base = r0 + t * CH
      pltpu.sync_copy(rows_v, acc.at[pl.ds(base, CH)])
      if with_cnt:
        pltpu.sync_copy(zv_v, cacc.at[pl.ds(base, CH)])
    if with_cnt:
      pltpu.sync_copy(ones, ones_v)
    pltpu.sync_copy(srcw.at[wid], src_v)
    pltpu.sync_copy(dstw.at[wid], dst_v)
    plsc.subcore_barrier()

    @pl.loop(0, K)
    def _edge_chunk(j):
      pltpu.async_copy(table.at[src_v.at[j]], rows_v, sem).wait()
      pltpu.sync_copy(rows_v, acc.at[dst_v.at[j]], add=True)
      if with_cnt:
        pltpu.sync_copy(ones_v, cacc.at[dst_v.at[j]], add=True)

    plsc.subcore_barrier()

    for t in range(RPW // CH):
      base = r0 + t * CH
      pltpu.sync_copy(acc.at[pl.ds(base, CH)], rows_v)
      pltpu.sync_copy(rows_v, parts.at[cid, pl.ds(base, CH)])
      if with_cnt:
        pltpu.sync_copy(cacc.at[pl.ds(base, CH)], zv_v)
        pltpu.sync_copy(zv_v, cnts.at[cid, pl.ds(base, CH)])

  return pl.kernel(body, out_type=tuple(out_type), mesh=mesh,
                   scratch_types=scratch,
                   compiler_params=pltpu.CompilerParams(
                       use_tc_tiling_on_sc=False))


_seg_sum_cnt = _make_seg_sum(H, with_cnt=True)
_seg_sum_o = _make_seg_sum(O, with_cnt=False)


def _tc_project(xp, wl, wr):
  def body(x_ref, wl_ref, wr_ref, p_ref, r_ref):
    xb = x_ref[...]
    p_ref[...] = jnp.dot(xb, wl_ref[...], preferred_element_type=jnp.float32)
    r_ref[...] = jnp.dot(xb, wr_ref[...], preferred_element_type=jnp.float32)

  return pl.pallas_call(
      body,
      grid=(NP // RB,),
      in_specs=[
          pl.BlockSpec((RB, D), lambda i: (i, 0)),
          pl.BlockSpec((D, H), lambda i: (0, 0)),
          pl.BlockSpec((D, H), lambda i: (0, 0)),
      ],
      out_specs=[
          pl.BlockSpec((RB, H), lambda i: (i, 0)),
          pl.BlockSpec((RB, H), lambda i: (i, 0)),
      ],
      out_shape=[
          jax.ShapeDtypeStruct((NP, H), jnp.float32),
          jax.ShapeDtypeStruct((NP, H), jnp.float32),
      ],
  )(xp, wl, wr)


def _tc_mid(parts0, cntt, r0, alpha, bb, wcat):
  def body(pp_ref, cn_ref, r0_ref, al_ref, bb_ref, w_ref, p1_ref, r1_ref):
    agg = pp_ref[0] + pp_ref[1]
    cnt = jnp.maximum(cn_ref[:, 0:1] + cn_ref[:, 1:2], 1.0)
    mean = agg / cnt
    h = jnp.maximum((mean + r0_ref[...]) * al_ref[...] + bb_ref[...], 0.0)
    pr = jnp.dot(h, w_ref[...], preferred_element_type=jnp.float32)
    p1_ref[...] = pr[:, :O]
    r1_ref[...] = pr[:, O:]

  return pl.pallas_call(
      body,
      grid=(NP // RB,),
      in_specs=[
          pl.BlockSpec((NC, RB, H), lambda i: (0, i, 0)),
          pl.BlockSpec((RB, NC), lambda i: (i, 0)),
          pl.BlockSpec((RB, H), lambda i: (i, 0)),
          pl.BlockSpec((1, H), lambda i: (0, 0)),
          pl.BlockSpec((1, H), lambda i: (0, 0)),
          pl.BlockSpec((H, 2 * O), lambda i: (0, 0)),
      ],
      out_specs=[
          pl.BlockSpec((RB, O), lambda i: (i, 0)),
          pl.BlockSpec((RB, O), lambda i: (i, 0)),
      ],
      out_shape=[
          jax.ShapeDtypeStruct((NP, O), jnp.float32),
          jax.ShapeDtypeStruct((NP, O), jnp.float32),
      ],
  )(parts0, cntt, r0, alpha, bb, wcat)


def _tc_final(parts1, cntt, r1, b1):
  def body(pp_ref, cn_ref, r1_ref, b1_ref, out_ref):
    agg = pp_ref[0] + pp_ref[1]
    cnt = jnp.maximum(cn_ref[:, 0:1] + cn_ref[:, 1:2], 1.0)
    out_ref[...] = agg / cnt + r1_ref[...] + b1_ref[...]

  return pl.pallas_call(
      body,
      grid=(NP // RB,),
      in_specs=[
          pl.BlockSpec((NC, RB, O), lambda i: (0, i, 0)),
          pl.BlockSpec((RB, NC), lambda i: (i, 0)),
          pl.BlockSpec((RB, O), lambda i: (i, 0)),
          pl.BlockSpec((1, O), lambda i: (0, 0)),
      ],
      out_specs=pl.BlockSpec((RB, O), lambda i: (i, 0)),
      out_shape=jax.ShapeDtypeStruct((NP, O), jnp.float32),
  )(parts1, cntt, r1, b1)


def kernel(x, edge_index, Wl0, Wr0, b0, gamma0, beta0, Wl1, Wr1, b1):
  f32 = jnp.float32
  xp = jnp.pad(x, ((0, NP - N), (0, 0)))
  src = jnp.concatenate(
      [edge_index[0], jnp.zeros((EPAD - E,), jnp.int32)]).reshape(NW, K, CH)
  dst = jnp.concatenate(
      [edge_index[1], jnp.full((EPAD - E,), NP - 1, jnp.int32)]).reshape(NW, K, CH)
  ones = jnp.ones((CH,), f32)
  zvec = jnp.zeros((CH,), f32)
  zrows_h = jnp.zeros((CH, H), f32)
  zrows_o = jnp.zeros((CH, O), f32)

  p0, r0 = _tc_project(xp, Wl0, Wr0)
  parts0, cntp = _seg_sum_cnt(p0, src, dst, ones, zrows_h, zvec)
  cntt = cntp.T

  scale = 1.0 / jnp.sqrt(jnp.float32(1.0) + BN_EPS)
  alpha = (gamma0 * scale).reshape(1, H)
  bb = (b0 * gamma0 * scale + beta0).reshape(1, H)
  wcat = jnp.concatenate([Wl1, Wr1], axis=1)

  p1, r1 = _tc_mid(parts0, cntt, r0, alpha, bb, wcat)
  (parts1,) = _seg_sum_o(p1, src, dst, ones, zrows_o, zvec)
  out = _tc_final(parts1, cntt, r1, b1.reshape(1, O))
  return out[:N]
